# baseline (speedup 1.0000x reference)
import functools

import jax
import jax.numpy as jnp
from jax import lax
from jax.experimental import pallas as pl
from jax.experimental.pallas import tpu as pltpu
from jax.experimental.pallas import tpu_sc as plsc

_B = 16
_N = 10000
_KB = 128
_AGG_ROWS = 10016
_DUMMY = 10008
_NB = 10
_BN = _N // _NB
_F32 = jnp.float32


def _elu(v):
    return jnp.where(v > 0, v, jnp.exp(jnp.minimum(v, 0.0)) - 1.0)


@functools.cache
def _sc_agg(nb, nh, nc):
    mesh = plsc.VectorSubcoreMesh(core_axis_name="c", subcore_axis_name="s")

    def body(x_flat, src_h, dst_h, zeros_h, out, src_scr, dst_scr, gbuf_a,
             gbuf_b, agg, sem_a, sem_b):
        cid = lax.axis_index("c")
        tid = lax.axis_index("s")
        base = tid * 624

        def gather(b, buf, sem):
            return pltpu.async_copy(x_flat.at[src_scr.at[b]], buf, sem)

        def wait_gather(b, buf, sem):
            pltpu.make_async_copy(x_flat.at[src_scr.at[b]], buf, sem).wait()

        for c in range(nc):
            pltpu.sync_copy(zeros_h, gbuf_a)
            for off in range(0, 624, _KB):
                pltpu.sync_copy(gbuf_a.at[pl.ds(0, min(_KB, 624 - off))],
                                agg.at[pl.ds(base + off, min(_KB, 624 - off))])

            @pl.when(tid == 15)
            def _():
                pltpu.sync_copy(gbuf_a, agg.at[pl.ds(10000 - _KB, _KB)])

            plsc.subcore_barrier()

            for h in range(nh):
                pltpu.sync_copy(src_h.at[c, cid, tid, h], src_scr)
                pltpu.sync_copy(dst_h.at[0, cid, tid, h], dst_scr)
                gather(0, gbuf_a, sem_a)
                gather(1, gbuf_b, sem_b)

                def step(i, carry):
                    b = 2 * i
                    wait_gather(b, gbuf_a, sem_a)
                    pltpu.sync_copy(gbuf_a, agg.at[dst_scr.at[b]], add=True)
                    gather(b + 2, gbuf_a, sem_a)
                    wait_gather(b + 1, gbuf_b, sem_b)
                    pltpu.sync_copy(gbuf_b, agg.at[dst_scr.at[b + 1]],
                                    add=True)
                    gather(b + 3, gbuf_b, sem_b)
                    return carry

                lax.fori_loop(0, nb // 2, step, 0)
                wait_gather(nb, gbuf_a, sem_a)
                wait_gather(nb + 1, gbuf_b, sem_b)
            plsc.subcore_barrier()

            pltpu.sync_copy(agg.at[pl.ds(base, 624)],
                            out.at[cid, c, pl.ds(base, 624)])

            @pl.when(tid == 15)
            def _():
                pltpu.sync_copy(agg.at[pl.ds(9984, 16)],
                                out.at[cid, c, pl.ds(9984, 16)])

            if c < nc - 1:
                plsc.subcore_barrier()

    return pl.kernel(
        body,
        out_type=jax.ShapeDtypeStruct((2, nc, _N, 128), _F32),
        mesh=mesh,
        scratch_types=[
            pltpu.VMEM((nb + 2, _KB), jnp.int32),
            pltpu.VMEM((nb + 2, _KB), jnp.int32),
            pltpu.VMEM((_KB, 128), _F32),
            pltpu.VMEM((_KB, 128), _F32),
            pltpu.VMEM_SHARED((_AGG_ROWS, 128), _F32),
            pltpu.SemaphoreType.DMA,
            pltpu.SemaphoreType.DMA,
        ],
    )


def _conv_body(p_ref, x_ref, wrel_ref, wroot_ref, b_ref, out_ref, acc):
    ci = pl.program_id(2)
    nc_in = pl.num_programs(2)

    @pl.when(ci == 0)
    def _():
        acc[...] = jnp.zeros_like(acc)

    aggb = p_ref[0, 0] + p_ref[1, 0]
    acc[...] += (jnp.dot(aggb, wrel_ref[...], preferred_element_type=_F32)
                 + jnp.dot(x_ref[0], wroot_ref[...],
                           preferred_element_type=_F32))

    @pl.when(ci == nc_in - 1)
    def _():
        out_ref[0] = _elu(acc[...] + b_ref[...])


def _conv_tc(P, X, wrel, wroot, b, nc_in, nc_out):
    return pl.pallas_call(
        _conv_body,
        grid=(_NB, nc_out, nc_in),
        in_specs=[
            pl.BlockSpec((2, 1, _BN, 128), lambda n, co, ci: (0, ci, n, 0)),
            pl.BlockSpec((1, _BN, 128), lambda n, co, ci: (ci, n, 0)),
            pl.BlockSpec((128, 128), lambda n, co, ci: (ci, co)),
            pl.BlockSpec((128, 128), lambda n, co, ci: (ci, co)),
            pl.BlockSpec((1, 128), lambda n, co, ci: (0, co)),
        ],
        out_specs=pl.BlockSpec((1, _BN, 128), lambda n, co, ci: (co, n, 0)),
        out_shape=jax.ShapeDtypeStruct((nc_out, _N, 128), _F32),
        scratch_shapes=[pltpu.VMEM((_BN, 128), _F32)],
    )(P, X, wrel, wroot, b.reshape(1, -1))


def _pre3_body(x_ref, wrel_ref, wroot_ref, b_ref, y_ref, r_ref, accy, accr):
    ci = pl.program_id(2)
    nc_in = pl.num_programs(2)

    @pl.when(ci == 0)
    def _():
        accy[...] = jnp.zeros_like(accy)
        accr[...] = jnp.zeros_like(accr)

    accy[...] += jnp.dot(x_ref[0], wrel_ref[...], preferred_element_type=_F32)
    accr[...] += jnp.dot(x_ref[0], wroot_ref[...],
                         preferred_element_type=_F32)

    @pl.when(ci == nc_in - 1)
    def _():
        y_ref[0] = accy[...]
        r_ref[0] = accr[...] + b_ref[...]


def _pre3_tc(X, wrel, wroot, b, nc_in, nc_out):
    return pl.pallas_call(
        _pre3_body,
        grid=(_NB, nc_out, nc_in),
        in_specs=[
            pl.BlockSpec((1, _BN, 128), lambda n, co, ci: (ci, n, 0)),
            pl.BlockSpec((128, 128), lambda n, co, ci: (ci, co)),
            pl.BlockSpec((128, 128), lambda n, co, ci: (ci, co)),
            pl.BlockSpec((1, 128), lambda n, co, ci: (0, co)),
        ],
        out_specs=[
            pl.BlockSpec((1, _BN, 128), lambda n, co, ci: (co, n, 0)),
            pl.BlockSpec((1, _BN, 128), lambda n, co, ci: (co, n, 0)),
        ],
        out_shape=[
            jax.ShapeDtypeStruct((nc_out, _N, 128), _F32),
            jax.ShapeDtypeStruct((nc_out, _N, 128), _F32),
        ],
        scratch_shapes=[pltpu.VMEM((_BN, 128), _F32),
                        pltpu.VMEM((_BN, 128), _F32)],
    )(X, wrel, wroot, b.reshape(1, -1))


def _pool_body(p_ref, r_ref, batch_ref, sums_ref, cnt_ref, accs, accc):
    co = pl.program_id(0)
    n = pl.program_id(1)

    @pl.when(n == 0)
    def _():
        accs[...] = jnp.zeros_like(accs)
        accc[...] = jnp.zeros_like(accc)

    x3 = _elu(p_ref[0, 0] + p_ref[1, 0] + r_ref[0])
    bt = batch_ref[0]
    seg = lax.broadcasted_iota(jnp.int32, (_B, _BN), 0)
    S = (seg == jnp.broadcast_to(bt, (_B, _BN))).astype(_F32)
    accs[...] += jnp.dot(S, x3, preferred_element_type=_F32)

    @pl.when(co == 0)
    def _():
        accc[...] += jnp.broadcast_to(
            jnp.sum(S, axis=1, keepdims=True), (_B, 128))

    @pl.when(n == _NB - 1)
    def _():
        sums_ref[...] = accs[...]

        @pl.when(co == 0)
        def _():
            cnt_ref[...] = accc[...]


def _pool_tc(P, R, batch3d, nc_out):
    return pl.pallas_call(
        _pool_body,
        grid=(nc_out, _NB),
        in_specs=[
            pl.BlockSpec((2, 1, _BN, 128), lambda co, n: (0, co, n, 0)),
            pl.BlockSpec((1, _BN, 128), lambda co, n: (co, n, 0)),
            pl.BlockSpec((1, 1, _BN), lambda co, n: (n, 0, 0)),
        ],
        out_specs=[
            pl.BlockSpec((_B, 128), lambda co, n: (0, co)),
            pl.BlockSpec((_B, 128), lambda co, n: (0, 0)),
        ],
        out_shape=[
            jax.ShapeDtypeStruct((_B, 128 * nc_out), _F32),
            jax.ShapeDtypeStruct((_B, 128), _F32),
        ],
        scratch_shapes=[pltpu.VMEM((_B, 128), _F32),
                        pltpu.VMEM((_B, 128), _F32)],
    )(P, R, batch3d)


def _head_body(gs_ref, gc_ref, ss_ref, sc_ref, pt_ref, w1_ref, b1_ref,
               w2_ref, b2_ref, w3_ref, b3_ref, out_ref):
    x1 = gs_ref[...] / jnp.maximum(gc_ref[:, 0:1], 1.0)
    x2 = ss_ref[...] / jnp.maximum(sc_ref[:, 0:1], 1.0)
    x = jnp.concatenate([x1, x2, pt_ref[...]], axis=-1)
    h = jnp.maximum(jnp.dot(x, w1_ref[...], preferred_element_type=_F32)
                    + b1_ref[...], 0.0)
    h = jnp.maximum(jnp.dot(h, w2_ref[...], preferred_element_type=_F32)
                    + b2_ref[...], 0.0)
    out_ref[...] = (jnp.dot(h, w3_ref[...], preferred_element_type=_F32)
                    + b3_ref[...])


def _head_tc(gs, gc, ss, sc_, point, lin_params):
    (w1, b1), (w2, b2), (w3, b3) = lin_params
    return pl.pallas_call(
        _head_body,
        out_shape=jax.ShapeDtypeStruct((_B, w3.shape[1]), _F32),
    )(gs, gc, ss, sc_, point, w1, b1.reshape(1, -1), w2, b2.reshape(1, -1),
      w3, b3.reshape(1, -1))


def _prep_edges(edge_index, n_edges, nh):
    src = edge_index[0].astype(jnp.int32)
    dst = edge_index[1].astype(jnp.int32)
    nb = -(-n_edges // 32 // nh // _KB)
    nb += nb % 2
    e_pad = 32 * nh * nb * _KB
    srcp = jnp.concatenate(
        [src, jnp.zeros((e_pad - n_edges,), jnp.int32)]
    ).reshape(2, 16, nh, nb, _KB)
    dstp = jnp.concatenate(
        [dst, jnp.full((e_pad - n_edges,), _DUMMY, jnp.int32)]
    ).reshape(2, 16, nh, nb, _KB)
    src_tail = jnp.zeros((2, 16, nh, 2, _KB), jnp.int32)
    dst_tail = jnp.full((2, 16, nh, 2, _KB), _DUMMY, jnp.int32)
    srcp = jnp.concatenate([srcp, src_tail], axis=3)
    dst_h = jnp.concatenate([dstp, dst_tail], axis=3)[None]
    src_hs = {}
    for nc in (1, 2, 3):
        offs = (jnp.arange(nc, dtype=jnp.int32) * _N).reshape(
            nc, 1, 1, 1, 1, 1)
        src_hs[nc] = srcp[None] + offs
    return src_hs, dst_h, nb


def _conv_net(x0, edge_index, batch, params, n_edges, zeros128):
    nh = 2 if n_edges > 200000 else 1
    src_hs, dst_h, nb = _prep_edges(edge_index, n_edges, nh)
    (wr1, wo1, b1), (wr2, wo2, b2), (wr3, wo3, b3) = params

    X = x0.reshape(1, _N, 128)
    P1 = _sc_agg(nb, nh, 1)(x0, src_hs[1], dst_h, zeros128)
    X2 = _conv_tc(P1, X, wr1, wo1, b1, 1, 2)

    P2 = _sc_agg(nb, nh, 2)(X2.reshape(2 * _N, 128), src_hs[2], dst_h, zeros128)
    X3 = _conv_tc(P2, X2, wr2, wo2, b2, 2, 4)

    Y, R = _pre3_tc(X3, wr3, wo3, b3, 4, 3)
    P3 = _sc_agg(nb, nh, 3)(Y.reshape(3 * _N, 128), src_hs[3], dst_h, zeros128)

    batch3d = batch.astype(jnp.int32).reshape(_NB, 1, _BN)
    return _pool_tc(P3, R, batch3d, 3)


def kernel(graph_x, graph_edge_index, graph_batch, subgraph_x,
           subgraph_edge_index, subgraph_batch, point, g_params, s_params,
           lin_params):
    zeros128 = jnp.zeros((128, 128), _F32)
    gs, gc = _conv_net(graph_x, graph_edge_index, graph_batch, g_params,
                       320000, zeros128)
    ss, sc_ = _conv_net(subgraph_x, subgraph_edge_index, subgraph_batch,
                        s_params, 160000, zeros128)
    return _head_tc(gs, gc, ss, sc_, point, lin_params)


# speedup vs baseline: 1.7865x; 1.7865x over previous
import functools

import jax
import jax.numpy as jnp
from jax import lax
from jax.experimental import pallas as pl
from jax.experimental.pallas import tpu as pltpu
from jax.experimental.pallas import tpu_sc as plsc

_B = 16
_N = 10000
_KB = 128
_AGG_ROWS = 10016
_DUMMY = 10008
_NB = 10
_BN = _N // _NB
_F32 = jnp.float32


def _elu(v):
    return jnp.where(v > 0, v, jnp.exp(jnp.minimum(v, 0.0)) - 1.0)


@functools.cache
def _sc_agg(nb, nh, nc):
    mesh = plsc.VectorSubcoreMesh(core_axis_name="c", subcore_axis_name="s")

    def body(x_flat, src_h, dst_h, zeros_h, out, src_scr, dst_scr, gbuf_a,
             agg, sem_a):
        cid = lax.axis_index("c")
        tid = lax.axis_index("s")
        base = tid * 624

        def gather(b, buf, sem):
            return pltpu.async_copy(x_flat.at[src_scr.at[b]], buf, sem)

        for c in range(nc):
            pltpu.sync_copy(zeros_h, gbuf_a)
            for off in range(0, 624, _KB):
                pltpu.sync_copy(gbuf_a.at[pl.ds(0, min(_KB, 624 - off))],
                                agg.at[pl.ds(base + off, min(_KB, 624 - off))])

            @pl.when(tid == 15)
            def _():
                pltpu.sync_copy(gbuf_a, agg.at[pl.ds(10000 - _KB, _KB)])

            plsc.subcore_barrier()

            for h in range(nh):
                pltpu.sync_copy(src_h.at[c, cid, tid, h], src_scr)
                pltpu.sync_copy(dst_h.at[0, cid, tid, h], dst_scr)

                def step(b, carry):
                    gather(b, gbuf_a, sem_a).wait()
                    pltpu.sync_copy(gbuf_a, agg.at[dst_scr.at[b]], add=True)
                    return carry

                lax.fori_loop(0, nb, step, 0)
            plsc.subcore_barrier()

            pltpu.sync_copy(agg.at[pl.ds(base, 624)],
                            out.at[cid, c, pl.ds(base, 624)])

            @pl.when(tid == 15)
            def _():
                pltpu.sync_copy(agg.at[pl.ds(9984, 16)],
                                out.at[cid, c, pl.ds(9984, 16)])

            if c < nc - 1:
                plsc.subcore_barrier()

    return pl.kernel(
        body,
        out_type=jax.ShapeDtypeStruct((2, nc, _N, 128), _F32),
        mesh=mesh,
        scratch_types=[
            pltpu.VMEM((nb + 2, _KB), jnp.int32),
            pltpu.VMEM((nb + 2, _KB), jnp.int32),
            pltpu.VMEM((_KB, 128), _F32),
            pltpu.VMEM_SHARED((_AGG_ROWS, 128), _F32),
            pltpu.SemaphoreType.DMA,
        ],
    )


def _conv_body(p_ref, x_ref, wrel_ref, wroot_ref, b_ref, out_ref, acc):
    ci = pl.program_id(2)
    nc_in = pl.num_programs(2)

    @pl.when(ci == 0)
    def _():
        acc[...] = jnp.zeros_like(acc)

    aggb = p_ref[0, 0] + p_ref[1, 0]
    acc[...] += (jnp.dot(aggb, wrel_ref[...], preferred_element_type=_F32)
                 + jnp.dot(x_ref[0], wroot_ref[...],
                           preferred_element_type=_F32))

    @pl.when(ci == nc_in - 1)
    def _():
        out_ref[0] = _elu(acc[...] + b_ref[...])


def _conv_tc(P, X, wrel, wroot, b, nc_in, nc_out):
    return pl.pallas_call(
        _conv_body,
        grid=(_NB, nc_out, nc_in),
        in_specs=[
            pl.BlockSpec((2, 1, _BN, 128), lambda n, co, ci: (0, ci, n, 0)),
            pl.BlockSpec((1, _BN, 128), lambda n, co, ci: (ci, n, 0)),
            pl.BlockSpec((128, 128), lambda n, co, ci: (ci, co)),
            pl.BlockSpec((128, 128), lambda n, co, ci: (ci, co)),
            pl.BlockSpec((1, 128), lambda n, co, ci: (0, co)),
        ],
        out_specs=pl.BlockSpec((1, _BN, 128), lambda n, co, ci: (co, n, 0)),
        out_shape=jax.ShapeDtypeStruct((nc_out, _N, 128), _F32),
        scratch_shapes=[pltpu.VMEM((_BN, 128), _F32)],
    )(P, X, wrel, wroot, b.reshape(1, -1))


def _pre3_body(x_ref, wrel_ref, wroot_ref, b_ref, y_ref, r_ref, accy, accr):
    ci = pl.program_id(2)
    nc_in = pl.num_programs(2)

    @pl.when(ci == 0)
    def _():
        accy[...] = jnp.zeros_like(accy)
        accr[...] = jnp.zeros_like(accr)

    accy[...] += jnp.dot(x_ref[0], wrel_ref[...], preferred_element_type=_F32)
    accr[...] += jnp.dot(x_ref[0], wroot_ref[...],
                         preferred_element_type=_F32)

    @pl.when(ci == nc_in - 1)
    def _():
        y_ref[0] = accy[...]
        r_ref[0] = accr[...] + b_ref[...]


def _pre3_tc(X, wrel, wroot, b, nc_in, nc_out):
    return pl.pallas_call(
        _pre3_body,
        grid=(_NB, nc_out, nc_in),
        in_specs=[
            pl.BlockSpec((1, _BN, 128), lambda n, co, ci: (ci, n, 0)),
            pl.BlockSpec((128, 128), lambda n, co, ci: (ci, co)),
            pl.BlockSpec((128, 128), lambda n, co, ci: (ci, co)),
            pl.BlockSpec((1, 128), lambda n, co, ci: (0, co)),
        ],
        out_specs=[
            pl.BlockSpec((1, _BN, 128), lambda n, co, ci: (co, n, 0)),
            pl.BlockSpec((1, _BN, 128), lambda n, co, ci: (co, n, 0)),
        ],
        out_shape=[
            jax.ShapeDtypeStruct((nc_out, _N, 128), _F32),
            jax.ShapeDtypeStruct((nc_out, _N, 128), _F32),
        ],
        scratch_shapes=[pltpu.VMEM((_BN, 128), _F32),
                        pltpu.VMEM((_BN, 128), _F32)],
    )(X, wrel, wroot, b.reshape(1, -1))


def _pool_body(p_ref, r_ref, batch_ref, sums_ref, cnt_ref, accs, accc):
    co = pl.program_id(0)
    n = pl.program_id(1)

    @pl.when(n == 0)
    def _():
        accs[...] = jnp.zeros_like(accs)
        accc[...] = jnp.zeros_like(accc)

    x3 = _elu(p_ref[0, 0] + p_ref[1, 0] + r_ref[0])
    bt = batch_ref[0]
    seg = lax.broadcasted_iota(jnp.int32, (_B, _BN), 0)
    S = (seg == jnp.broadcast_to(bt, (_B, _BN))).astype(_F32)
    accs[...] += jnp.dot(S, x3, preferred_element_type=_F32)

    @pl.when(co == 0)
    def _():
        accc[...] += jnp.broadcast_to(
            jnp.sum(S, axis=1, keepdims=True), (_B, 128))

    @pl.when(n == _NB - 1)
    def _():
        sums_ref[...] = accs[...]

        @pl.when(co == 0)
        def _():
            cnt_ref[...] = accc[...]


def _pool_tc(P, R, batch3d, nc_out):
    return pl.pallas_call(
        _pool_body,
        grid=(nc_out, _NB),
        in_specs=[
            pl.BlockSpec((2, 1, _BN, 128), lambda co, n: (0, co, n, 0)),
            pl.BlockSpec((1, _BN, 128), lambda co, n: (co, n, 0)),
            pl.BlockSpec((1, 1, _BN), lambda co, n: (n, 0, 0)),
        ],
        out_specs=[
            pl.BlockSpec((_B, 128), lambda co, n: (0, co)),
            pl.BlockSpec((_B, 128), lambda co, n: (0, 0)),
        ],
        out_shape=[
            jax.ShapeDtypeStruct((_B, 128 * nc_out), _F32),
            jax.ShapeDtypeStruct((_B, 128), _F32),
        ],
        scratch_shapes=[pltpu.VMEM((_B, 128), _F32),
                        pltpu.VMEM((_B, 128), _F32)],
    )(P, R, batch3d)


def _head_body(gs_ref, gc_ref, ss_ref, sc_ref, pt_ref, w1_ref, b1_ref,
               w2_ref, b2_ref, w3_ref, b3_ref, out_ref):
    x1 = gs_ref[...] / jnp.maximum(gc_ref[:, 0:1], 1.0)
    x2 = ss_ref[...] / jnp.maximum(sc_ref[:, 0:1], 1.0)
    x = jnp.concatenate([x1, x2, pt_ref[...]], axis=-1)
    h = jnp.maximum(jnp.dot(x, w1_ref[...], preferred_element_type=_F32)
                    + b1_ref[...], 0.0)
    h = jnp.maximum(jnp.dot(h, w2_ref[...], preferred_element_type=_F32)
                    + b2_ref[...], 0.0)
    out_ref[...] = (jnp.dot(h, w3_ref[...], preferred_element_type=_F32)
                    + b3_ref[...])


def _head_tc(gs, gc, ss, sc_, point, lin_params):
    (w1, b1), (w2, b2), (w3, b3) = lin_params
    return pl.pallas_call(
        _head_body,
        out_shape=jax.ShapeDtypeStruct((_B, w3.shape[1]), _F32),
    )(gs, gc, ss, sc_, point, w1, b1.reshape(1, -1), w2, b2.reshape(1, -1),
      w3, b3.reshape(1, -1))


def _prep_edges(edge_index, n_edges, nh):
    src = edge_index[0].astype(jnp.int32)
    dst = edge_index[1].astype(jnp.int32)
    nb = -(-n_edges // 32 // nh // _KB)
    nb += nb % 2
    e_pad = 32 * nh * nb * _KB
    srcp = jnp.concatenate(
        [src, jnp.zeros((e_pad - n_edges,), jnp.int32)]
    ).reshape(2, 16, nh, nb, _KB)
    dstp = jnp.concatenate(
        [dst, jnp.full((e_pad - n_edges,), _DUMMY, jnp.int32)]
    ).reshape(2, 16, nh, nb, _KB)
    src_tail = jnp.zeros((2, 16, nh, 2, _KB), jnp.int32)
    dst_tail = jnp.full((2, 16, nh, 2, _KB), _DUMMY, jnp.int32)
    srcp = jnp.concatenate([srcp, src_tail], axis=3)
    dst_h = jnp.concatenate([dstp, dst_tail], axis=3)[None]
    src_hs = {}
    for nc in (1, 2, 3):
        offs = (jnp.arange(nc, dtype=jnp.int32) * _N).reshape(
            nc, 1, 1, 1, 1, 1)
        src_hs[nc] = srcp[None] + offs
    return src_hs, dst_h, nb


def kernel(graph_x, graph_edge_index, graph_batch, subgraph_x,
           subgraph_edge_index, subgraph_batch, point, g_params, s_params,
           lin_params):
    zeros128 = jnp.zeros((128, 128), _F32)
    nets = []
    for x0, ei, bt, params, n_edges in (
            (graph_x, graph_edge_index, graph_batch, g_params, 320000),
            (subgraph_x, subgraph_edge_index, subgraph_batch, s_params,
             160000)):
        nh = 2 if n_edges > 200000 else 1
        src_hs, dst_h, nb = _prep_edges(ei, n_edges, nh)
        nets.append(dict(x0=x0, batch=bt, params=params, nb=nb, nh=nh,
                         src=src_hs, dst=dst_h))

    for n in nets:
        n["P"] = _sc_agg(n["nb"], n["nh"], 1)(
            n["x0"], n["src"][1], n["dst"], zeros128)
    for n in nets:
        wr, wo, b = n["params"][0]
        n["X"] = _conv_tc(n["P"], n["x0"].reshape(1, _N, 128), wr, wo, b,
                          1, 2)
    for n in nets:
        n["P"] = _sc_agg(n["nb"], n["nh"], 2)(
            n["X"].reshape(2 * _N, 128), n["src"][2], n["dst"], zeros128)
    for n in nets:
        wr, wo, b = n["params"][1]
        n["X"] = _conv_tc(n["P"], n["X"], wr, wo, b, 2, 4)
    for n in nets:
        wr, wo, b = n["params"][2]
        n["Y"], n["R"] = _pre3_tc(n["X"], wr, wo, b, 4, 3)
    for n in nets:
        n["P"] = _sc_agg(n["nb"], n["nh"], 3)(
            n["Y"].reshape(3 * _N, 128), n["src"][3], n["dst"], zeros128)
    for n in nets:
        b3d = n["batch"].astype(jnp.int32).reshape(_NB, 1, _BN)
        n["sums"], n["cnt"] = _pool_tc(n["P"], n["R"], b3d, 3)
    return _head_tc(nets[0]["sums"], nets[0]["cnt"], nets[1]["sums"],
                    nets[1]["cnt"], point, lin_params)


# speedup vs baseline: 1.8288x; 1.0236x over previous
import functools

import jax
import jax.numpy as jnp
from jax import lax
from jax.experimental import pallas as pl
from jax.experimental.pallas import tpu as pltpu
from jax.experimental.pallas import tpu_sc as plsc

_B = 16
_N = 10000
_KB = 128
_AGG_ROWS = 10016
_DUMMY = 10008
_NB = 10
_BN = _N // _NB
_F32 = jnp.float32


def _elu(v):
    return jnp.where(v > 0, v, jnp.exp(jnp.minimum(v, 0.0)) - 1.0)


@functools.cache
def _sc_agg(nb, nh, nc):
    mesh = plsc.VectorSubcoreMesh(core_axis_name="c", subcore_axis_name="s")

    def body(x_flat, src_h, dst_h, zeros_h, out, src_scr, dst_scr, gbuf_a,
             agg, sem_a):
        cid = lax.axis_index("c")
        tid = lax.axis_index("s")
        base = tid * 624

        def gather(b, buf, sem):
            return pltpu.async_copy(x_flat.at[src_scr.at[b]], buf, sem)

        for c in range(nc):
            pltpu.sync_copy(zeros_h, gbuf_a)
            for off in range(0, 624, _KB):
                pltpu.sync_copy(gbuf_a.at[pl.ds(0, min(_KB, 624 - off))],
                                agg.at[pl.ds(base + off, min(_KB, 624 - off))])

            @pl.when(tid == 15)
            def _():
                pltpu.sync_copy(gbuf_a, agg.at[pl.ds(10000 - _KB, _KB)])

            plsc.subcore_barrier()

            for h in range(nh):
                pltpu.sync_copy(src_h.at[c, cid, tid, h], src_scr)
                pltpu.sync_copy(dst_h.at[0, cid, tid, h], dst_scr)

                def step(b, carry):
                    gather(b, gbuf_a, sem_a).wait()
                    pltpu.sync_copy(gbuf_a, agg.at[dst_scr.at[b]], add=True)
                    return carry

                lax.fori_loop(0, nb, step, 0)
            plsc.subcore_barrier()

            pltpu.sync_copy(agg.at[pl.ds(base, 624)],
                            out.at[cid, c, pl.ds(base, 624)])

            @pl.when(tid == 15)
            def _():
                pltpu.sync_copy(agg.at[pl.ds(9984, 16)],
                                out.at[cid, c, pl.ds(9984, 16)])

            if c < nc - 1:
                plsc.subcore_barrier()

    return pl.kernel(
        body,
        out_type=jax.ShapeDtypeStruct((2, nc, _N, 128), _F32),
        mesh=mesh,
        scratch_types=[
            pltpu.VMEM((nb + 2, _KB), jnp.int32),
            pltpu.VMEM((nb + 2, _KB), jnp.int32),
            pltpu.VMEM((_KB, 128), _F32),
            pltpu.VMEM_SHARED((_AGG_ROWS, 128), _F32),
            pltpu.SemaphoreType.DMA,
        ],
    )


def _conv_body(p_ref, x_ref, wrel_ref, wroot_ref, b_ref, out_ref, acc):
    ci = pl.program_id(2)
    nc_in = pl.num_programs(2)

    @pl.when(ci == 0)
    def _():
        acc[...] = jnp.zeros_like(acc)

    aggb = p_ref[0, 0] + p_ref[1, 0]
    acc[...] += (jnp.dot(aggb, wrel_ref[...], preferred_element_type=_F32)
                 + jnp.dot(x_ref[0], wroot_ref[...],
                           preferred_element_type=_F32))

    @pl.when(ci == nc_in - 1)
    def _():
        out_ref[0] = _elu(acc[...] + b_ref[...])


def _conv_tc(P, X, wrel, wroot, b, nc_in, nc_out):
    return pl.pallas_call(
        _conv_body,
        grid=(_NB, nc_out, nc_in),
        in_specs=[
            pl.BlockSpec((2, 1, _BN, 128), lambda n, co, ci: (0, ci, n, 0)),
            pl.BlockSpec((1, _BN, 128), lambda n, co, ci: (ci, n, 0)),
            pl.BlockSpec((128, 128), lambda n, co, ci: (ci, co)),
            pl.BlockSpec((128, 128), lambda n, co, ci: (ci, co)),
            pl.BlockSpec((1, 128), lambda n, co, ci: (0, co)),
        ],
        out_specs=pl.BlockSpec((1, _BN, 128), lambda n, co, ci: (co, n, 0)),
        out_shape=jax.ShapeDtypeStruct((nc_out, _N, 128), _F32),
        scratch_shapes=[pltpu.VMEM((_BN, 128), _F32)],
    )(P, X, wrel, wroot, b.reshape(1, -1))


def _pre3_body(x_ref, wrel_ref, wroot_ref, b_ref, y_ref, r_ref, accy, accr):
    ci = pl.program_id(2)
    nc_in = pl.num_programs(2)

    @pl.when(ci == 0)
    def _():
        accy[...] = jnp.zeros_like(accy)
        accr[...] = jnp.zeros_like(accr)

    accy[...] += jnp.dot(x_ref[0], wrel_ref[...], preferred_element_type=_F32)
    accr[...] += jnp.dot(x_ref[0], wroot_ref[...],
                         preferred_element_type=_F32)

    @pl.when(ci == nc_in - 1)
    def _():
        y_ref[0] = accy[...]
        r_ref[0] = accr[...] + b_ref[...]


def _pre3_tc(X, wrel, wroot, b, nc_in, nc_out):
    return pl.pallas_call(
        _pre3_body,
        grid=(_NB, nc_out, nc_in),
        in_specs=[
            pl.BlockSpec((1, _BN, 128), lambda n, co, ci: (ci, n, 0)),
            pl.BlockSpec((128, 128), lambda n, co, ci: (ci, co)),
            pl.BlockSpec((128, 128), lambda n, co, ci: (ci, co)),
            pl.BlockSpec((1, 128), lambda n, co, ci: (0, co)),
        ],
        out_specs=[
            pl.BlockSpec((1, _BN, 128), lambda n, co, ci: (co, n, 0)),
            pl.BlockSpec((1, _BN, 128), lambda n, co, ci: (co, n, 0)),
        ],
        out_shape=[
            jax.ShapeDtypeStruct((nc_out, _N, 128), _F32),
            jax.ShapeDtypeStruct((nc_out, _N, 128), _F32),
        ],
        scratch_shapes=[pltpu.VMEM((_BN, 128), _F32),
                        pltpu.VMEM((_BN, 128), _F32)],
    )(X, wrel, wroot, b.reshape(1, -1))


def _pool_body(p_ref, r_ref, batch_ref, sums_ref, cnt_ref, accs, accc):
    co = pl.program_id(0)
    n = pl.program_id(1)

    @pl.when(n == 0)
    def _():
        accs[...] = jnp.zeros_like(accs)
        accc[...] = jnp.zeros_like(accc)

    x3 = _elu(p_ref[0, 0] + p_ref[1, 0] + r_ref[0])
    bt = batch_ref[0]
    seg = lax.broadcasted_iota(jnp.int32, (_B, _BN), 0)
    S = (seg == jnp.broadcast_to(bt, (_B, _BN))).astype(_F32)
    accs[...] += jnp.dot(S, x3, preferred_element_type=_F32)

    @pl.when(co == 0)
    def _():
        accc[...] += jnp.broadcast_to(
            jnp.sum(S, axis=1, keepdims=True), (_B, 128))

    @pl.when(n == _NB - 1)
    def _():
        sums_ref[...] = accs[...]

        @pl.when(co == 0)
        def _():
            cnt_ref[...] = accc[...]


def _pool_tc(P, R, batch3d, nc_out):
    return pl.pallas_call(
        _pool_body,
        grid=(nc_out, _NB),
        in_specs=[
            pl.BlockSpec((2, 1, _BN, 128), lambda co, n: (0, co, n, 0)),
            pl.BlockSpec((1, _BN, 128), lambda co, n: (co, n, 0)),
            pl.BlockSpec((1, 1, _BN), lambda co, n: (n, 0, 0)),
        ],
        out_specs=[
            pl.BlockSpec((_B, 128), lambda co, n: (0, co)),
            pl.BlockSpec((_B, 128), lambda co, n: (0, 0)),
        ],
        out_shape=[
            jax.ShapeDtypeStruct((_B, 128 * nc_out), _F32),
            jax.ShapeDtypeStruct((_B, 128), _F32),
        ],
        scratch_shapes=[pltpu.VMEM((_B, 128), _F32),
                        pltpu.VMEM((_B, 128), _F32)],
    )(P, R, batch3d)


def _head_body(gs_ref, gc_ref, ss_ref, sc_ref, pt_ref, w1_ref, b1_ref,
               w2_ref, b2_ref, w3_ref, b3_ref, out_ref):
    x1 = gs_ref[...] / jnp.maximum(gc_ref[:, 0:1], 1.0)
    x2 = ss_ref[...] / jnp.maximum(sc_ref[:, 0:1], 1.0)
    x = jnp.concatenate([x1, x2, pt_ref[...]], axis=-1)
    h = jnp.maximum(jnp.dot(x, w1_ref[...], preferred_element_type=_F32)
                    + b1_ref[...], 0.0)
    h = jnp.maximum(jnp.dot(h, w2_ref[...], preferred_element_type=_F32)
                    + b2_ref[...], 0.0)
    out_ref[...] = (jnp.dot(h, w3_ref[...], preferred_element_type=_F32)
                    + b3_ref[...])


def _head_tc(gs, gc, ss, sc_, point, lin_params):
    (w1, b1), (w2, b2), (w3, b3) = lin_params
    return pl.pallas_call(
        _head_body,
        out_shape=jax.ShapeDtypeStruct((_B, w3.shape[1]), _F32),
    )(gs, gc, ss, sc_, point, w1, b1.reshape(1, -1), w2, b2.reshape(1, -1),
      w3, b3.reshape(1, -1))


def _prep_edges(edge_index, n_edges, nh):
    src = edge_index[0].astype(jnp.int32)
    dst = edge_index[1].astype(jnp.int32)
    nb = -(-n_edges // 32 // nh // _KB)
    nb += nb % 2
    e_pad = 32 * nh * nb * _KB
    srcp = jnp.concatenate(
        [src, jnp.zeros((e_pad - n_edges,), jnp.int32)]
    ).reshape(2, 16, nh, nb, _KB)
    dstp = jnp.concatenate(
        [dst, jnp.full((e_pad - n_edges,), _DUMMY, jnp.int32)]
    ).reshape(2, 16, nh, nb, _KB)
    src_tail = jnp.zeros((2, 16, nh, 2, _KB), jnp.int32)
    dst_tail = jnp.full((2, 16, nh, 2, _KB), _DUMMY, jnp.int32)
    srcp = jnp.concatenate([srcp, src_tail], axis=3)
    dst_h = jnp.concatenate([dstp, dst_tail], axis=3)[None]
    src_hs = {}
    for nc in (1, 2, 3):
        offs = (jnp.arange(nc, dtype=jnp.int32) * _N).reshape(
            nc, 1, 1, 1, 1, 1)
        src_hs[nc] = srcp[None] + offs
    return src_hs, dst_h, nb


def kernel(graph_x, graph_edge_index, graph_batch, subgraph_x,
           subgraph_edge_index, subgraph_batch, point, g_params, s_params,
           lin_params):
    zeros128 = jnp.zeros((128, 128), _F32)
    nets = []
    for x0, ei, bt, params, n_edges in (
            (graph_x, graph_edge_index, graph_batch, g_params, 320000),
            (subgraph_x, subgraph_edge_index, subgraph_batch, s_params,
             160000)):
        nh = 2 if n_edges > 200000 else 1
        src_hs, dst_h, nb = _prep_edges(ei, n_edges, nh)
        nets.append(dict(x0=x0, batch=bt, params=params, nb=nb, nh=nh,
                         src=src_hs, dst=dst_h))

    for n in nets:
        n["P"] = _sc_agg(n["nb"], n["nh"], 1)(
            n["x0"], n["src"][1], n["dst"], zeros128)
        wr, wo, b = n["params"][0]
        n["X"] = _conv_tc(n["P"], n["x0"].reshape(1, _N, 128), wr, wo, b,
                          1, 2)
        n["P"] = _sc_agg(n["nb"], n["nh"], 2)(
            n["X"].reshape(2 * _N, 128), n["src"][2], n["dst"], zeros128)
        wr, wo, b = n["params"][1]
        n["X"] = _conv_tc(n["P"], n["X"], wr, wo, b, 2, 4)
        wr, wo, b = n["params"][2]
        n["Y"], n["R"] = _pre3_tc(n["X"], wr, wo, b, 4, 3)
        n["P"] = _sc_agg(n["nb"], n["nh"], 3)(
            n["Y"].reshape(3 * _N, 128), n["src"][3], n["dst"], zeros128)
        b3d = n["batch"].astype(jnp.int32).reshape(_NB, 1, _BN)
        n["sums"], n["cnt"] = _pool_tc(n["P"], n["R"], b3d, 3)
    return _head_tc(nets[0]["sums"], nets[0]["cnt"], nets[1]["sums"],
                    nets[1]["cnt"], point, lin_params)


# speedup vs baseline: 2.2911x; 1.2528x over previous
import functools

import jax
import jax.numpy as jnp
from jax import lax
from jax.experimental import pallas as pl
from jax.experimental.pallas import tpu as pltpu
from jax.experimental.pallas import tpu_sc as plsc

_B = 16
_N = 10000
_KB = 128
_AGG_ROWS = 10016
_DUMMY = 10008
_NB = 10
_BN = _N // _NB
_F32 = jnp.float32


def _elu(v):
    return jnp.where(v > 0, v, jnp.exp(jnp.minimum(v, 0.0)) - 1.0)


@functools.cache
def _sc_agg(nb, nh, nc):
    mesh = plsc.VectorSubcoreMesh(core_axis_name="c", subcore_axis_name="s")

    def body(x_flat, src_h, dst_h, zeros_h, out, src_scr, dst_scr, gbuf_a,
             agg, sem_a):
        cid = lax.axis_index("c")
        tid = lax.axis_index("s")
        base = tid * 624

        def gather(b, buf, sem):
            return pltpu.async_copy(x_flat.at[src_scr.at[b]], buf, sem)

        for c in range(nc):
            pltpu.sync_copy(zeros_h, gbuf_a)
            for off in range(0, 624, _KB):
                pltpu.sync_copy(gbuf_a.at[pl.ds(0, min(_KB, 624 - off))],
                                agg.at[pl.ds(base + off, min(_KB, 624 - off))])

            @pl.when(tid == 15)
            def _():
                pltpu.sync_copy(gbuf_a, agg.at[pl.ds(10000 - _KB, _KB)])

            plsc.subcore_barrier()

            for h in range(nh):
                pltpu.sync_copy(src_h.at[c, cid, tid, h], src_scr)
                pltpu.sync_copy(dst_h.at[0, cid, tid, h], dst_scr)

                def step(b, carry):
                    gather(b, gbuf_a, sem_a).wait()
                    pltpu.sync_copy(gbuf_a, agg.at[dst_scr.at[b]], add=True)
                    return carry

                lax.fori_loop(0, nb, step, 0)
            plsc.subcore_barrier()

            pltpu.sync_copy(agg.at[pl.ds(base, 624)],
                            out.at[cid, c, pl.ds(base, 624)])

            @pl.when(tid == 15)
            def _():
                pltpu.sync_copy(agg.at[pl.ds(9984, 16)],
                                out.at[cid, c, pl.ds(9984, 16)])

            if c < nc - 1:
                plsc.subcore_barrier()

    return pl.kernel(
        body,
        out_type=jax.ShapeDtypeStruct((2, nc, _N, 128), _F32),
        mesh=mesh,
        scratch_types=[
            pltpu.VMEM((nb, _KB), jnp.int32),
            pltpu.VMEM((nb, _KB), jnp.int32),
            pltpu.VMEM((_KB, 128), _F32),
            pltpu.VMEM_SHARED((_AGG_ROWS, 128), _F32),
            pltpu.SemaphoreType.DMA,
        ],
    )


def _conv_body(p_ref, x_ref, wrel_ref, wroot_ref, b_ref, out_ref, acc):
    ci = pl.program_id(2)
    nc_in = pl.num_programs(2)

    @pl.when(ci == 0)
    def _():
        acc[...] = jnp.zeros_like(acc)

    aggb = p_ref[0, 0] + p_ref[1, 0]
    acc[...] += (jnp.dot(aggb, wrel_ref[...], preferred_element_type=_F32)
                 + jnp.dot(x_ref[0], wroot_ref[...],
                           preferred_element_type=_F32))

    @pl.when(ci == nc_in - 1)
    def _():
        out_ref[0] = _elu(acc[...] + b_ref[...])


def _conv_tc(P, X, wrel, wroot, b, nc_in, nc_out):
    return pl.pallas_call(
        _conv_body,
        grid=(_NB, nc_out, nc_in),
        in_specs=[
            pl.BlockSpec((2, 1, _BN, 128), lambda n, co, ci: (0, ci, n, 0)),
            pl.BlockSpec((1, _BN, 128), lambda n, co, ci: (ci, n, 0)),
            pl.BlockSpec((128, 128), lambda n, co, ci: (ci, co)),
            pl.BlockSpec((128, 128), lambda n, co, ci: (ci, co)),
            pl.BlockSpec((1, 128), lambda n, co, ci: (0, co)),
        ],
        out_specs=pl.BlockSpec((1, _BN, 128), lambda n, co, ci: (co, n, 0)),
        out_shape=jax.ShapeDtypeStruct((nc_out, _N, 128), _F32),
        scratch_shapes=[pltpu.VMEM((_BN, 128), _F32)],
    )(P, X, wrel, wroot, b.reshape(1, -1))


def _pre3_body(x_ref, wrel_ref, wroot_ref, b_ref, y_ref, r_ref, accy, accr):
    ci = pl.program_id(2)
    nc_in = pl.num_programs(2)

    @pl.when(ci == 0)
    def _():
        accy[...] = jnp.zeros_like(accy)
        accr[...] = jnp.zeros_like(accr)

    accy[...] += jnp.dot(x_ref[0], wrel_ref[...], preferred_element_type=_F32)
    accr[...] += jnp.dot(x_ref[0], wroot_ref[...],
                         preferred_element_type=_F32)

    @pl.when(ci == nc_in - 1)
    def _():
        y_ref[0] = accy[...]
        r_ref[0] = accr[...] + b_ref[...]


def _pre3_tc(X, wrel, wroot, b, nc_in, nc_out):
    return pl.pallas_call(
        _pre3_body,
        grid=(_NB, nc_out, nc_in),
        in_specs=[
            pl.BlockSpec((1, _BN, 128), lambda n, co, ci: (ci, n, 0)),
            pl.BlockSpec((128, 128), lambda n, co, ci: (ci, co)),
            pl.BlockSpec((128, 128), lambda n, co, ci: (ci, co)),
            pl.BlockSpec((1, 128), lambda n, co, ci: (0, co)),
        ],
        out_specs=[
            pl.BlockSpec((1, _BN, 128), lambda n, co, ci: (co, n, 0)),
            pl.BlockSpec((1, _BN, 128), lambda n, co, ci: (co, n, 0)),
        ],
        out_shape=[
            jax.ShapeDtypeStruct((nc_out, _N, 128), _F32),
            jax.ShapeDtypeStruct((nc_out, _N, 128), _F32),
        ],
        scratch_shapes=[pltpu.VMEM((_BN, 128), _F32),
                        pltpu.VMEM((_BN, 128), _F32)],
    )(X, wrel, wroot, b.reshape(1, -1))


def _pool_body(p_ref, r_ref, batch_ref, sums_ref, cnt_ref, accs, accc):
    co = pl.program_id(0)
    n = pl.program_id(1)

    @pl.when(n == 0)
    def _():
        accs[...] = jnp.zeros_like(accs)
        accc[...] = jnp.zeros_like(accc)

    x3 = _elu(p_ref[0, 0] + p_ref[1, 0] + r_ref[0])
    bt = batch_ref[0]
    seg = lax.broadcasted_iota(jnp.int32, (_B, _BN), 0)
    S = (seg == jnp.broadcast_to(bt, (_B, _BN))).astype(_F32)
    accs[...] += jnp.dot(S, x3, preferred_element_type=_F32)

    @pl.when(co == 0)
    def _():
        accc[...] += jnp.broadcast_to(
            jnp.sum(S, axis=1, keepdims=True), (_B, 128))

    @pl.when(n == _NB - 1)
    def _():
        sums_ref[...] = accs[...]

        @pl.when(co == 0)
        def _():
            cnt_ref[...] = accc[...]


def _pool_tc(P, R, batch3d, nc_out):
    return pl.pallas_call(
        _pool_body,
        grid=(nc_out, _NB),
        in_specs=[
            pl.BlockSpec((2, 1, _BN, 128), lambda co, n: (0, co, n, 0)),
            pl.BlockSpec((1, _BN, 128), lambda co, n: (co, n, 0)),
            pl.BlockSpec((1, 1, _BN), lambda co, n: (n, 0, 0)),
        ],
        out_specs=[
            pl.BlockSpec((_B, 128), lambda co, n: (0, co)),
            pl.BlockSpec((_B, 128), lambda co, n: (0, 0)),
        ],
        out_shape=[
            jax.ShapeDtypeStruct((_B, 128 * nc_out), _F32),
            jax.ShapeDtypeStruct((_B, 128), _F32),
        ],
        scratch_shapes=[pltpu.VMEM((_B, 128), _F32),
                        pltpu.VMEM((_B, 128), _F32)],
    )(P, R, batch3d)


def _head_body(gs_ref, gc_ref, ss_ref, sc_ref, pt_ref, w1_ref, b1_ref,
               w2_ref, b2_ref, w3_ref, b3_ref, out_ref):
    x1 = gs_ref[...] / jnp.maximum(gc_ref[:, 0:1], 1.0)
    x2 = ss_ref[...] / jnp.maximum(sc_ref[:, 0:1], 1.0)
    x = jnp.concatenate([x1, x2, pt_ref[...]], axis=-1)
    h = jnp.maximum(jnp.dot(x, w1_ref[...], preferred_element_type=_F32)
                    + b1_ref[...], 0.0)
    h = jnp.maximum(jnp.dot(h, w2_ref[...], preferred_element_type=_F32)
                    + b2_ref[...], 0.0)
    out_ref[...] = (jnp.dot(h, w3_ref[...], preferred_element_type=_F32)
                    + b3_ref[...])


def _head_tc(gs, gc, ss, sc_, point, lin_params):
    (w1, b1), (w2, b2), (w3, b3) = lin_params
    return pl.pallas_call(
        _head_body,
        out_shape=jax.ShapeDtypeStruct((_B, w3.shape[1]), _F32),
    )(gs, gc, ss, sc_, point, w1, b1.reshape(1, -1), w2, b2.reshape(1, -1),
      w3, b3.reshape(1, -1))


def _prep_edges(edge_index, n_edges, nh):
    src = edge_index[0].astype(jnp.int32)
    dst = edge_index[1].astype(jnp.int32)
    nb = -(-n_edges // 32 // nh // _KB)
    e_pad = 32 * nh * nb * _KB
    npad = e_pad - n_edges
    pad_dst = _DUMMY + (jnp.arange(npad, dtype=jnp.int32) % 8)
    srcp = jnp.concatenate(
        [src, jnp.zeros((npad,), jnp.int32)]).reshape(2, 16, nh, nb, _KB)
    dst_h = jnp.concatenate(
        [dst, pad_dst]).reshape(2, 16, nh, nb, _KB)[None]
    src_hs = {}
    for nc in (1, 2, 3):
        offs = (jnp.arange(nc, dtype=jnp.int32) * _N).reshape(
            nc, 1, 1, 1, 1, 1)
        src_hs[nc] = srcp[None] + offs
    return src_hs, dst_h, nb


def kernel(graph_x, graph_edge_index, graph_batch, subgraph_x,
           subgraph_edge_index, subgraph_batch, point, g_params, s_params,
           lin_params):
    zeros128 = jnp.zeros((128, 128), _F32)
    nets = []
    for x0, ei, bt, params, n_edges in (
            (graph_x, graph_edge_index, graph_batch, g_params, 320000),
            (subgraph_x, subgraph_edge_index, subgraph_batch, s_params,
             160000)):
        nh = 1
        src_hs, dst_h, nb = _prep_edges(ei, n_edges, nh)
        nets.append(dict(x0=x0, batch=bt, params=params, nb=nb, nh=nh,
                         src=src_hs, dst=dst_h))

    for n in nets:
        n["P"] = _sc_agg(n["nb"], n["nh"], 1)(
            n["x0"], n["src"][1], n["dst"], zeros128)
        wr, wo, b = n["params"][0]
        n["X"] = _conv_tc(n["P"], n["x0"].reshape(1, _N, 128), wr, wo, b,
                          1, 2)
        n["P"] = _sc_agg(n["nb"], n["nh"], 2)(
            n["X"].reshape(2 * _N, 128), n["src"][2], n["dst"], zeros128)
        wr, wo, b = n["params"][1]
        n["X"] = _conv_tc(n["P"], n["X"], wr, wo, b, 2, 4)
        wr, wo, b = n["params"][2]
        n["Y"], n["R"] = _pre3_tc(n["X"], wr, wo, b, 4, 3)
        n["P"] = _sc_agg(n["nb"], n["nh"], 3)(
            n["Y"].reshape(3 * _N, 128), n["src"][3], n["dst"], zeros128)
        b3d = n["batch"].astype(jnp.int32).reshape(_NB, 1, _BN)
        n["sums"], n["cnt"] = _pool_tc(n["P"], n["R"], b3d, 3)
    return _head_tc(nets[0]["sums"], nets[0]["cnt"], nets[1]["sums"],
                    nets[1]["cnt"], point, lin_params)


# speedup vs baseline: 2.3712x; 1.0350x over previous
import functools

import jax
import jax.numpy as jnp
from jax import lax
from jax.experimental import pallas as pl
from jax.experimental.pallas import tpu as pltpu
from jax.experimental.pallas import tpu_sc as plsc

_B = 16
_N = 10000
_KB = 128
_AGG_ROWS = 10016
_DUMMY = 10008
_NB = 10
_BN = _N // _NB
_F32 = jnp.float32


def _elu(v):
    return jnp.where(v > 0, v, jnp.exp(jnp.minimum(v, 0.0)) - 1.0)


@functools.cache
def _sc_agg(nb, nh, nc):
    mesh = plsc.VectorSubcoreMesh(core_axis_name="c", subcore_axis_name="s")

    def body(x_flat, src_h, dst_h, zeros_h, out, src_scr, dst_scr, gbuf_a,
             agg, sem_a):
        cid = lax.axis_index("c")
        tid = lax.axis_index("s")
        base = tid * 624

        def gather(b, buf, sem):
            return pltpu.async_copy(x_flat.at[src_scr.at[b]], buf, sem)

        for c in range(nc):
            pltpu.sync_copy(zeros_h, gbuf_a)
            for off in range(0, 624, _KB):
                pltpu.sync_copy(gbuf_a.at[pl.ds(0, min(_KB, 624 - off))],
                                agg.at[pl.ds(base + off, min(_KB, 624 - off))])

            @pl.when(tid == 15)
            def _():
                pltpu.sync_copy(gbuf_a, agg.at[pl.ds(10000 - _KB, _KB)])

            plsc.subcore_barrier()

            for h in range(nh):
                pltpu.sync_copy(src_h.at[c, cid, tid, h], src_scr)
                pltpu.sync_copy(dst_h.at[0, cid, tid, h], dst_scr)

                def step(b, carry):
                    gather(b, gbuf_a, sem_a).wait()
                    pltpu.sync_copy(gbuf_a, agg.at[dst_scr.at[b]], add=True)
                    return carry

                lax.fori_loop(0, nb, step, 0)
            plsc.subcore_barrier()

            pltpu.sync_copy(agg.at[pl.ds(base, 624)],
                            out.at[cid, c, pl.ds(base, 624)])

            @pl.when(tid == 15)
            def _():
                pltpu.sync_copy(agg.at[pl.ds(9984, 16)],
                                out.at[cid, c, pl.ds(9984, 16)])

            if c < nc - 1:
                plsc.subcore_barrier()

    return pl.kernel(
        body,
        out_type=jax.ShapeDtypeStruct((2, nc, _N, 128), _F32),
        mesh=mesh,
        scratch_types=[
            pltpu.VMEM((nb, _KB), jnp.int32),
            pltpu.VMEM((nb, _KB), jnp.int32),
            pltpu.VMEM((_KB, 128), _F32),
            pltpu.VMEM_SHARED((_AGG_ROWS, 128), _F32),
            pltpu.SemaphoreType.DMA,
        ],
    )


def _make_conv_body(nc_in):
    def body(p_ref, x_ref, wrel_ref, wroot_ref, b_ref, out_ref):
        lhs = jnp.concatenate(
            [p_ref[0, i] + p_ref[1, i] for i in range(nc_in)]
            + [x_ref[i] for i in range(nc_in)], axis=1)
        w = jnp.concatenate([wrel_ref[...], wroot_ref[...]], axis=0)
        out_ref[0] = _elu(
            jnp.dot(lhs, w, preferred_element_type=_F32) + b_ref[...])
    return body


def _conv_tc(P, X, wrel, wroot, b, nc_in, nc_out):
    d_in = 128 * nc_in
    return pl.pallas_call(
        _make_conv_body(nc_in),
        grid=(_NB, nc_out),
        in_specs=[
            pl.BlockSpec((2, nc_in, _BN, 128), lambda n, co: (0, 0, n, 0)),
            pl.BlockSpec((nc_in, _BN, 128), lambda n, co: (0, n, 0)),
            pl.BlockSpec((d_in, 128), lambda n, co: (0, co)),
            pl.BlockSpec((d_in, 128), lambda n, co: (0, co)),
            pl.BlockSpec((1, 128), lambda n, co: (0, co)),
        ],
        out_specs=pl.BlockSpec((1, _BN, 128), lambda n, co: (co, n, 0)),
        out_shape=jax.ShapeDtypeStruct((nc_out, _N, 128), _F32),
    )(P, X, wrel, wroot, b.reshape(1, -1))


def _make_pre3_body(nc_in):
    def body(x_ref, wrel_ref, wroot_ref, b_ref, y_ref, r_ref):
        lhs = jnp.concatenate([x_ref[i] for i in range(nc_in)], axis=1)
        y_ref[0] = jnp.dot(lhs, wrel_ref[...], preferred_element_type=_F32)
        r_ref[0] = (jnp.dot(lhs, wroot_ref[...], preferred_element_type=_F32)
                    + b_ref[...])
    return body


def _pre3_tc(X, wrel, wroot, b, nc_in, nc_out):
    d_in = 128 * nc_in
    return pl.pallas_call(
        _make_pre3_body(nc_in),
        grid=(_NB, nc_out),
        in_specs=[
            pl.BlockSpec((nc_in, _BN, 128), lambda n, co: (0, n, 0)),
            pl.BlockSpec((d_in, 128), lambda n, co: (0, co)),
            pl.BlockSpec((d_in, 128), lambda n, co: (0, co)),
            pl.BlockSpec((1, 128), lambda n, co: (0, co)),
        ],
        out_specs=[
            pl.BlockSpec((1, _BN, 128), lambda n, co: (co, n, 0)),
            pl.BlockSpec((1, _BN, 128), lambda n, co: (co, n, 0)),
        ],
        out_shape=[
            jax.ShapeDtypeStruct((nc_out, _N, 128), _F32),
            jax.ShapeDtypeStruct((nc_out, _N, 128), _F32),
        ],
    )(X, wrel, wroot, b.reshape(1, -1))


def _pool_body(p_ref, r_ref, batch_ref, sums_ref, cnt_ref, accs, accc):
    co = pl.program_id(0)
    n = pl.program_id(1)

    @pl.when(n == 0)
    def _():
        accs[...] = jnp.zeros_like(accs)
        accc[...] = jnp.zeros_like(accc)

    x3 = _elu(p_ref[0, 0] + p_ref[1, 0] + r_ref[0])
    bt = batch_ref[0]
    seg = lax.broadcasted_iota(jnp.int32, (_B, _BN), 0)
    S = (seg == jnp.broadcast_to(bt, (_B, _BN))).astype(_F32)
    accs[...] += jnp.dot(S, x3, preferred_element_type=_F32)

    @pl.when(co == 0)
    def _():
        accc[...] += jnp.broadcast_to(
            jnp.sum(S, axis=1, keepdims=True), (_B, 128))

    @pl.when(n == _NB - 1)
    def _():
        sums_ref[...] = accs[...]

        @pl.when(co == 0)
        def _():
            cnt_ref[...] = accc[...]


def _pool_tc(P, R, batch3d, nc_out):
    return pl.pallas_call(
        _pool_body,
        grid=(nc_out, _NB),
        in_specs=[
            pl.BlockSpec((2, 1, _BN, 128), lambda co, n: (0, co, n, 0)),
            pl.BlockSpec((1, _BN, 128), lambda co, n: (co, n, 0)),
            pl.BlockSpec((1, 1, _BN), lambda co, n: (n, 0, 0)),
        ],
        out_specs=[
            pl.BlockSpec((_B, 128), lambda co, n: (0, co)),
            pl.BlockSpec((_B, 128), lambda co, n: (0, 0)),
        ],
        out_shape=[
            jax.ShapeDtypeStruct((_B, 128 * nc_out), _F32),
            jax.ShapeDtypeStruct((_B, 128), _F32),
        ],
        scratch_shapes=[pltpu.VMEM((_B, 128), _F32),
                        pltpu.VMEM((_B, 128), _F32)],
    )(P, R, batch3d)


def _head_body(gs_ref, gc_ref, ss_ref, sc_ref, pt_ref, w1_ref, b1_ref,
               w2_ref, b2_ref, w3_ref, b3_ref, out_ref):
    x1 = gs_ref[...] / jnp.maximum(gc_ref[:, 0:1], 1.0)
    x2 = ss_ref[...] / jnp.maximum(sc_ref[:, 0:1], 1.0)
    x = jnp.concatenate([x1, x2, pt_ref[...]], axis=-1)
    h = jnp.maximum(jnp.dot(x, w1_ref[...], preferred_element_type=_F32)
                    + b1_ref[...], 0.0)
    h = jnp.maximum(jnp.dot(h, w2_ref[...], preferred_element_type=_F32)
                    + b2_ref[...], 0.0)
    out_ref[...] = (jnp.dot(h, w3_ref[...], preferred_element_type=_F32)
                    + b3_ref[...])


def _head_tc(gs, gc, ss, sc_, point, lin_params):
    (w1, b1), (w2, b2), (w3, b3) = lin_params
    return pl.pallas_call(
        _head_body,
        out_shape=jax.ShapeDtypeStruct((_B, w3.shape[1]), _F32),
    )(gs, gc, ss, sc_, point, w1, b1.reshape(1, -1), w2, b2.reshape(1, -1),
      w3, b3.reshape(1, -1))


def _prep_edges(edge_index, n_edges, nh):
    src = edge_index[0].astype(jnp.int32)
    dst = edge_index[1].astype(jnp.int32)
    nb = -(-n_edges // 32 // nh // _KB)
    e_pad = 32 * nh * nb * _KB
    npad = e_pad - n_edges
    pad_dst = _DUMMY + (jnp.arange(npad, dtype=jnp.int32) % 8)
    srcp = jnp.concatenate(
        [src, jnp.zeros((npad,), jnp.int32)]).reshape(2, 16, nh, nb, _KB)
    dst_h = jnp.concatenate(
        [dst, pad_dst]).reshape(2, 16, nh, nb, _KB)[None]
    src_hs = {}
    for nc in (1, 2, 3):
        offs = (jnp.arange(nc, dtype=jnp.int32) * _N).reshape(
            nc, 1, 1, 1, 1, 1)
        src_hs[nc] = srcp[None] + offs
    return src_hs, dst_h, nb


def kernel(graph_x, graph_edge_index, graph_batch, subgraph_x,
           subgraph_edge_index, subgraph_batch, point, g_params, s_params,
           lin_params):
    zeros128 = jnp.zeros((128, 128), _F32)
    nets = []
    for x0, ei, bt, params, n_edges in (
            (graph_x, graph_edge_index, graph_batch, g_params, 320000),
            (subgraph_x, subgraph_edge_index, subgraph_batch, s_params,
             160000)):
        nh = 1
        src_hs, dst_h, nb = _prep_edges(ei, n_edges, nh)
        nets.append(dict(x0=x0, batch=bt, params=params, nb=nb, nh=nh,
                         src=src_hs, dst=dst_h))

    for n in nets:
        n["P"] = _sc_agg(n["nb"], n["nh"], 1)(
            n["x0"], n["src"][1], n["dst"], zeros128)
        wr, wo, b = n["params"][0]
        n["X"] = _conv_tc(n["P"], n["x0"].reshape(1, _N, 128), wr, wo, b,
                          1, 2)
        n["P"] = _sc_agg(n["nb"], n["nh"], 2)(
            n["X"].reshape(2 * _N, 128), n["src"][2], n["dst"], zeros128)
        wr, wo, b = n["params"][1]
        n["X"] = _conv_tc(n["P"], n["X"], wr, wo, b, 2, 4)
        wr, wo, b = n["params"][2]
        n["Y"], n["R"] = _pre3_tc(n["X"], wr, wo, b, 4, 3)
        n["P"] = _sc_agg(n["nb"], n["nh"], 3)(
            n["Y"].reshape(3 * _N, 128), n["src"][3], n["dst"], zeros128)
        b3d = n["batch"].astype(jnp.int32).reshape(_NB, 1, _BN)
        n["sums"], n["cnt"] = _pool_tc(n["P"], n["R"], b3d, 3)
    return _head_tc(nets[0]["sums"], nets[0]["cnt"], nets[1]["sums"],
                    nets[1]["cnt"], point, lin_params)


# speedup vs baseline: 2.3736x; 1.0010x over previous
import functools

import jax
import jax.numpy as jnp
from jax import lax
from jax.experimental import pallas as pl
from jax.experimental.pallas import tpu as pltpu
from jax.experimental.pallas import tpu_sc as plsc

_B = 16
_N = 10000
_KB = 128
_AGG_ROWS = 10016
_DUMMY = 10008
_NB = 10
_BN = _N // _NB
_F32 = jnp.float32


def _elu(v):
    return jnp.where(v > 0, v, jnp.exp(jnp.minimum(v, 0.0)) - 1.0)


@functools.cache
def _sc_agg(nb, nh, nc):
    mesh = plsc.VectorSubcoreMesh(core_axis_name="c", subcore_axis_name="s")

    def body(x_flat, src_h, dst_h, zeros_h, out, src_scr, dst_scr, gbuf_a,
             agg, sem_a):
        cid = lax.axis_index("c")
        tid = lax.axis_index("s")
        base = tid * 624

        def gather(b, buf, sem):
            return pltpu.async_copy(x_flat.at[src_scr.at[b]], buf, sem)

        for c in range(nc):
            pltpu.sync_copy(zeros_h, gbuf_a)
            for off in range(0, 624, _KB):
                pltpu.sync_copy(gbuf_a.at[pl.ds(0, min(_KB, 624 - off))],
                                agg.at[pl.ds(base + off, min(_KB, 624 - off))])

            @pl.when(tid == 15)
            def _():
                pltpu.sync_copy(gbuf_a, agg.at[pl.ds(10000 - _KB, _KB)])

            plsc.subcore_barrier()

            for h in range(nh):
                pltpu.sync_copy(src_h.at[c, cid, tid, h], src_scr)
                pltpu.sync_copy(dst_h.at[0, cid, tid, h], dst_scr)

                def step(b, carry):
                    gather(b, gbuf_a, sem_a).wait()
                    pltpu.sync_copy(gbuf_a, agg.at[dst_scr.at[b]], add=True)
                    return carry

                lax.fori_loop(0, nb, step, 0)
            plsc.subcore_barrier()

            pltpu.sync_copy(agg.at[pl.ds(base, 624)],
                            out.at[cid, c, pl.ds(base, 624)])

            @pl.when(tid == 15)
            def _():
                pltpu.sync_copy(agg.at[pl.ds(9984, 16)],
                                out.at[cid, c, pl.ds(9984, 16)])

            if c < nc - 1:
                plsc.subcore_barrier()

    return pl.kernel(
        body,
        out_type=jax.ShapeDtypeStruct((2, nc, _N, 128), _F32),
        mesh=mesh,
        scratch_types=[
            pltpu.VMEM((nb, _KB), jnp.int32),
            pltpu.VMEM((nb, _KB), jnp.int32),
            pltpu.VMEM((_KB, 128), _F32),
            pltpu.VMEM_SHARED((_AGG_ROWS, 128), _F32),
            pltpu.SemaphoreType.DMA,
        ],
    )


def _make_conv_body(nc_in):
    def body(p_ref, x_ref, wrel_ref, wroot_ref, b_ref, out_ref):
        lhs = jnp.concatenate(
            [p_ref[0, i] + p_ref[1, i] for i in range(nc_in)]
            + [x_ref[i] for i in range(nc_in)], axis=1)
        w = jnp.concatenate([wrel_ref[...], wroot_ref[...]], axis=0)
        out_ref[0] = _elu(
            jnp.dot(lhs, w, preferred_element_type=_F32) + b_ref[...])
    return body


def _conv_tc(P, X, wrel, wroot, b, nc_in, nc_out):
    d_in = 128 * nc_in
    return pl.pallas_call(
        _make_conv_body(nc_in),
        grid=(_NB, nc_out),
        in_specs=[
            pl.BlockSpec((2, nc_in, _BN, 128), lambda n, co: (0, 0, n, 0)),
            pl.BlockSpec((nc_in, _BN, 128), lambda n, co: (0, n, 0)),
            pl.BlockSpec((d_in, 128), lambda n, co: (0, co)),
            pl.BlockSpec((d_in, 128), lambda n, co: (0, co)),
            pl.BlockSpec((1, 128), lambda n, co: (0, co)),
        ],
        out_specs=pl.BlockSpec((1, _BN, 128), lambda n, co: (co, n, 0)),
        out_shape=jax.ShapeDtypeStruct((nc_out, _N, 128), _F32),
    )(P, X, wrel, wroot, b.reshape(1, -1))


def _make_pre3_body(nc_in):
    def body(x_ref, wrel_ref, wroot_ref, b_ref, y_ref, r_ref):
        lhs = jnp.concatenate([x_ref[i] for i in range(nc_in)], axis=1)
        y_ref[0] = jnp.dot(lhs, wrel_ref[...], preferred_element_type=_F32)
        r_ref[0] = (jnp.dot(lhs, wroot_ref[...], preferred_element_type=_F32)
                    + b_ref[...])
    return body


def _pre3_tc(X, wrel, wroot, b, nc_in, nc_out):
    d_in = 128 * nc_in
    return pl.pallas_call(
        _make_pre3_body(nc_in),
        grid=(_NB, nc_out),
        in_specs=[
            pl.BlockSpec((nc_in, _BN, 128), lambda n, co: (0, n, 0)),
            pl.BlockSpec((d_in, 128), lambda n, co: (0, co)),
            pl.BlockSpec((d_in, 128), lambda n, co: (0, co)),
            pl.BlockSpec((1, 128), lambda n, co: (0, co)),
        ],
        out_specs=[
            pl.BlockSpec((1, _BN, 128), lambda n, co: (co, n, 0)),
            pl.BlockSpec((1, _BN, 128), lambda n, co: (co, n, 0)),
        ],
        out_shape=[
            jax.ShapeDtypeStruct((nc_out, _N, 128), _F32),
            jax.ShapeDtypeStruct((nc_out, _N, 128), _F32),
        ],
    )(X, wrel, wroot, b.reshape(1, -1))


def _pool_body(p_ref, r_ref, batch_ref, sums_ref, cnt_ref, accs, accc):
    co = pl.program_id(0)
    n = pl.program_id(1)

    @pl.when(n == 0)
    def _():
        accs[...] = jnp.zeros_like(accs)
        accc[...] = jnp.zeros_like(accc)

    x3 = _elu(p_ref[0, 0] + p_ref[1, 0] + r_ref[0])
    bt = batch_ref[0]
    seg = lax.broadcasted_iota(jnp.int32, (_B, _BN), 0)
    S = (seg == jnp.broadcast_to(bt, (_B, _BN))).astype(_F32)
    accs[...] += jnp.dot(S, x3, preferred_element_type=_F32)

    @pl.when(co == 0)
    def _():
        accc[...] += jnp.broadcast_to(
            jnp.sum(S, axis=1, keepdims=True), (_B, 128))

    @pl.when(n == _NB - 1)
    def _():
        sums_ref[...] = accs[...]

        @pl.when(co == 0)
        def _():
            cnt_ref[...] = accc[...]


def _pool_tc(P, R, batch3d, nc_out):
    return pl.pallas_call(
        _pool_body,
        grid=(nc_out, _NB),
        in_specs=[
            pl.BlockSpec((2, 1, _BN, 128), lambda co, n: (0, co, n, 0)),
            pl.BlockSpec((1, _BN, 128), lambda co, n: (co, n, 0)),
            pl.BlockSpec((1, 1, _BN), lambda co, n: (n, 0, 0)),
        ],
        out_specs=[
            pl.BlockSpec((_B, 128), lambda co, n: (0, co)),
            pl.BlockSpec((_B, 128), lambda co, n: (0, 0)),
        ],
        out_shape=[
            jax.ShapeDtypeStruct((_B, 128 * nc_out), _F32),
            jax.ShapeDtypeStruct((_B, 128), _F32),
        ],
        scratch_shapes=[pltpu.VMEM((_B, 128), _F32),
                        pltpu.VMEM((_B, 128), _F32)],
    )(P, R, batch3d)


def _head_body(gs_ref, gc_ref, ss_ref, sc_ref, pt_ref, w1_ref, b1_ref,
               w2_ref, b2_ref, w3_ref, b3_ref, out_ref):
    x1 = gs_ref[...] / jnp.maximum(gc_ref[:, 0:1], 1.0)
    x2 = ss_ref[...] / jnp.maximum(sc_ref[:, 0:1], 1.0)
    x = jnp.concatenate([x1, x2, pt_ref[...]], axis=-1)
    h = jnp.maximum(jnp.dot(x, w1_ref[...], preferred_element_type=_F32)
                    + b1_ref[...], 0.0)
    h = jnp.maximum(jnp.dot(h, w2_ref[...], preferred_element_type=_F32)
                    + b2_ref[...], 0.0)
    out_ref[...] = (jnp.dot(h, w3_ref[...], preferred_element_type=_F32)
                    + b3_ref[...])


def _head_tc(gs, gc, ss, sc_, point, lin_params):
    (w1, b1), (w2, b2), (w3, b3) = lin_params
    return pl.pallas_call(
        _head_body,
        out_shape=jax.ShapeDtypeStruct((_B, w3.shape[1]), _F32),
    )(gs, gc, ss, sc_, point, w1, b1.reshape(1, -1), w2, b2.reshape(1, -1),
      w3, b3.reshape(1, -1))


def _prep_edges(edge_index, n_edges, nh):
    src = edge_index[0].astype(jnp.int32)
    dst = edge_index[1].astype(jnp.int32)
    nb = -(-n_edges // 32 // nh // _KB)
    e_pad = 32 * nh * nb * _KB
    npad = e_pad - n_edges
    pad_dst = _DUMMY + (jnp.arange(npad, dtype=jnp.int32) % 8)
    srcp = jnp.concatenate(
        [src, jnp.zeros((npad,), jnp.int32)]).reshape(2, 16, nh, nb, _KB)
    dst_h = jnp.concatenate(
        [dst, pad_dst]).reshape(2, 16, nh, nb, _KB)[None]
    src_hs = {}
    for nc in (1, 2, 3):
        offs = (jnp.arange(nc, dtype=jnp.int32) * _N).reshape(
            nc, 1, 1, 1, 1, 1)
        src_hs[nc] = srcp[None] + offs
    return src_hs, dst_h, nb


def kernel(graph_x, graph_edge_index, graph_batch, subgraph_x,
           subgraph_edge_index, subgraph_batch, point, g_params, s_params,
           lin_params):
    zeros128 = jnp.zeros((128, 128), _F32)
    nets = []
    for x0, ei, bt, params, n_edges in (
            (graph_x, graph_edge_index, graph_batch, g_params, 320000),
            (subgraph_x, subgraph_edge_index, subgraph_batch, s_params,
             160000)):
        nh = 1
        src_hs, dst_h, nb = _prep_edges(ei, n_edges, nh)
        nets.append(dict(x0=x0, batch=bt, params=params, nb=nb, nh=nh,
                         src=src_hs, dst=dst_h))

    for n in nets:
        n["P"] = _sc_agg(n["nb"], n["nh"], 1)(
            n["x0"], n["src"][1], n["dst"], zeros128)
    for n in nets:
        wr, wo, b = n["params"][0]
        n["X"] = _conv_tc(n["P"], n["x0"].reshape(1, _N, 128), wr, wo, b,
                          1, 2)
    for n in nets:
        n["P"] = _sc_agg(n["nb"], n["nh"], 2)(
            n["X"].reshape(2 * _N, 128), n["src"][2], n["dst"], zeros128)
    for n in nets:
        wr, wo, b = n["params"][1]
        n["X"] = _conv_tc(n["P"], n["X"], wr, wo, b, 2, 4)
    for n in nets:
        wr, wo, b = n["params"][2]
        n["Y"], n["R"] = _pre3_tc(n["X"], wr, wo, b, 4, 3)
    for n in nets:
        n["P"] = _sc_agg(n["nb"], n["nh"], 3)(
            n["Y"].reshape(3 * _N, 128), n["src"][3], n["dst"], zeros128)
    for n in nets:
        b3d = n["batch"].astype(jnp.int32).reshape(_NB, 1, _BN)
        n["sums"], n["cnt"] = _pool_tc(n["P"], n["R"], b3d, 3)
    return _head_tc(nets[0]["sums"], nets[0]["cnt"], nets[1]["sums"],
                    nets[1]["cnt"], point, lin_params)
